# baseline (speedup 1.0000x reference)
import functools

import jax
import jax.numpy as jnp
from jax import lax
from jax.experimental import pallas as pl
from jax.experimental.pallas import tpu as pltpu
from jax.experimental.pallas import tpu_sc as plsc

_B, _S, _H = 4, 8192, 1024
_N = _B * _S
_NC, _NS = 2, 16
_NW = _NC * _NS
_TOKW = _N // _NW
_CHUNK = 16
_NCH = _TOKW // _CHUNK
_LANES = 16
_NSLOT = 3

_mesh = plsc.VectorSubcoreMesh(core_axis_name="c", subcore_axis_name="s")


@functools.partial(
    pl.kernel,
    out_type=jax.ShapeDtypeStruct((_N, _H), jnp.float32),
    mesh=_mesh,
    scratch_types=[
        pltpu.VMEM((_NCH, _CHUNK), jnp.int32),
        pltpu.VMEM((_NCH, _CHUNK), jnp.int32),
    ] + [pltpu.VMEM((2 * _CHUNK, _H), jnp.float32)] * _NSLOT
      + [pltpu.SemaphoreType.DMA] * (2 * _NSLOT),
)
def _embed(ids_hbm, pos_hbm, wt_hbm, pt_hbm, out_hbm,
           widx, pidx, buf0, buf1, buf2,
           semg0, semst0, semg1, semst1, semg2, semst2):
    wid = lax.axis_index("s") * _NC + lax.axis_index("c")
    pltpu.sync_copy(ids_hbm.at[wid], widx)
    pltpu.sync_copy(pos_hbm.at[wid], pidx)

    slots = ((buf0, semg0, semst0),
             (buf1, semg1, semst1),
             (buf2, semg2, semst2))

    def fire(cc, buf, semg):
        pltpu.async_copy(wt_hbm.at[widx.at[cc]], buf.at[pl.ds(0, _CHUNK)], semg)
        pltpu.async_copy(pt_hbm.at[pidx.at[cc]], buf.at[pl.ds(_CHUNK, _CHUNK)],
                         semg)

    def segment(c, k, first):
        buf, semg, semst = slots[k]
        nbuf, nsemg, nsemst = slots[(k + 2) % _NSLOT]

        pltpu.make_async_copy(wt_hbm.at[pl.ds(0, 2 * _CHUNK)], buf, semg).wait()

        if first:
            fire(2, nbuf, nsemg)
        else:
            pltpu.make_async_copy(nbuf.at[pl.ds(0, _CHUNK)],
                                  out_hbm.at[pl.ds(0, _CHUNK)], nsemst).wait()

            @pl.when(c < _NCH - 2)
            def _fire_next():
                fire(c + 2, nbuf, nsemg)

        row0 = wid * _TOKW + c * _CHUNK
        half = _CHUNK // 2
        for q in range(2):
            @pl.loop(q * half, (q + 1) * half)
            def _rows(r):
                for j in range(_H // _LANES):
                    sl = pl.ds(j * _LANES, _LANES)
                    buf[r, sl] += buf[_CHUNK + r, sl]

            pltpu.async_copy(buf.at[pl.ds(q * half, half)],
                             out_hbm.at[pl.ds(row0 + q * half, half)], semst)

    fire(0, buf0, semg0)
    fire(1, buf1, semg1)

    segment(0, 0, first=True)

    @pl.loop(1, _NCH, step=_NSLOT)
    def _ring(c0):
        for k in range(_NSLOT):
            segment(c0 + k, (1 + k) % _NSLOT, first=False)

    pltpu.make_async_copy(buf0.at[pl.ds(0, _CHUNK)],
                          out_hbm.at[pl.ds(0, _CHUNK)], semst0).wait()


@jax.jit
def kernel(input_ids, position_ids, word_table, pos_table):
    ids = input_ids.astype(jnp.int32).reshape(_NW, _NCH, _CHUNK)
    pos = position_ids.astype(jnp.int32).reshape(_NW, _NCH, _CHUNK)
    out = _embed(ids, pos, word_table, pos_table)
    return out.reshape(_B, _S, _H)
